# baseline (speedup 1.0000x reference)
import functools
import math

import jax
import jax.numpy as jnp
from jax import lax
from jax.experimental import pallas as pl
from jax.experimental.pallas import tpu as pltpu
from jax.experimental.pallas import tpu_sc as plsc

N = 20000
NQ = 16
MS = (1, 3, 5, 7)
KS = tuple(m * NQ for m in MS)
LOFF = (0, 256, 512, 768)
OUT_D = 1024
BR = 16
NBLK = N // BR
NW = 32
CG = tuple(1.0 / math.sqrt(2 * l + 1) for l in range(4))

_mesh = plsc.VectorSubcoreMesh(core_axis_name="c", subcore_axis_name="s")

_IN_SCRATCH = [
    pltpu.VMEM((BR, KS[l]), jnp.float32) for l in range(4)
]


@functools.partial(
    pl.kernel,
    mesh=_mesh,
    compiler_params=pltpu.CompilerParams(needs_layout_passes=False),
    out_type=jax.ShapeDtypeStruct((N, OUT_D), jnp.float32),
    scratch_types=[
        *_IN_SCRATCH,
        *_IN_SCRATCH,
        pltpu.VMEM((NQ, NQ), jnp.int32),
        pltpu.VMEM((BR, OUT_D), jnp.float32),
        pltpu.VMEM((BR, OUT_D), jnp.float32),
        pltpu.SemaphoreType.DMA,
        pltpu.SemaphoreType.DMA,
        pltpu.SemaphoreType.DMA,
        pltpu.SemaphoreType.DMA,
    ],
)
def _ps_kernel(v0, v1, v2, v3, out,
               a0, a1, a2, a3, c0, c1, c2, c3, idxtab, ob0, ob1,
               si0, si1, so0, so1):
    cid = lax.axis_index("c")
    sid = lax.axis_index("s")
    wid = sid * 2 + cid
    nblk_w = 39 + (wid < (NBLK - NW * (NBLK // NW))).astype(jnp.int32)

    vs = (v0, v1, v2, v3)
    ins = ((a0, a1, a2, a3), (c0, c1, c2, c3))
    obs = (ob0, ob1)
    sin = (si0, si1)
    sout = (so0, so1)

    def blk_r0(b):
        return (wid + b * NW) * BR

    def issue_in(b, s):
        for l in range(4):
            pltpu.async_copy(vs[l].at[pl.ds(blk_r0(b), BR)], ins[s][l], sin[s])

    def wait_in(b, s):
        for l in range(4):
            pltpu.make_async_copy(
                vs[l].at[pl.ds(blk_r0(b), BR)], ins[s][l], sin[s]
            ).wait()

    for q in range(NQ):
        idxtab[q] = jnp.full((NQ,), q, jnp.int32)

    def compute(s):
        bufs = ins[s]
        ob = obs[s]

        @plsc.parallel_loop(0, BR, unroll=3)
        def rbody(r):
            idxq = [idxtab[q] for q in range(NQ)]
            for l in range(4):
                M = MS[l]
                raw = [bufs[l][r, pl.ds(mm * NQ, NQ)] for mm in range(M)]
                if CG[l] != 1.0:
                    scv = [v * CG[l] for v in raw]
                else:
                    scv = raw
                npair = (M // 2) if l >= 2 else 0
                praw = [
                    plsc.pack(raw[2 * k], raw[2 * k + 1],
                              format=plsc.PackFormat.INTERLEAVED)
                    for k in range(npair)
                ]
                pw = [
                    plsc.bitcast(
                        plsc.pack(scv[2 * k], scv[2 * k + 1],
                                  format=plsc.PackFormat.INTERLEAVED),
                        jnp.int32,
                    )
                    for k in range(npair)
                ]
                for q in range(NQ):
                    terms = []
                    if npair:
                        pterms = [
                            plsc.bitcast(
                                jnp.take_along_axis(pw[k], idxq[q], axis=0),
                                jnp.bfloat16,
                            ) * praw[k]
                            for k in range(npair)
                        ]
                        pacc = pterms[0]
                        for k in range(1, npair):
                            pacc = pacc + pterms[k]
                        au, bu = plsc.unpack(
                            pacc, format=plsc.PackFormat.INTERLEAVED
                        )
                        terms = [au, bu]
                    for mm in range(2 * npair, M):
                        terms.append(
                            jnp.take_along_axis(scv[mm], idxq[q], axis=0)
                            * raw[mm]
                        )
                    while len(terms) > 1:
                        terms = [
                            terms[k] + terms[k + 1] if k + 1 < len(terms)
                            else terms[k]
                            for k in range(0, len(terms), 2)
                        ]
                    ob[r, pl.ds(LOFF[l] + q * NQ, NQ)] = terms[0]

    def issue_out(b, s):
        pltpu.async_copy(obs[s], out.at[pl.ds(blk_r0(b), BR)], sout[s])

    def wait_out(b, s):
        pltpu.make_async_copy(
            obs[s], out.at[pl.ds(blk_r0(b), BR)], sout[s]
        ).wait()

    issue_in(0, 0)

    def pair_body(i2, carry):
        e = 2 * i2
        o = e + 1

        @pl.when(o < nblk_w)
        def _():
            issue_in(o, 1)

        wait_in(e, 0)

        @pl.when(i2 > 0)
        def _():
            wait_out(e - 2, 0)

        compute(0)
        issue_out(e, 0)

        @pl.when(o < nblk_w)
        def _():
            @pl.when(o + 1 < nblk_w)
            def _():
                issue_in(o + 1, 0)

            wait_in(o, 1)

            @pl.when(i2 > 0)
            def _():
                wait_out(o - 2, 1)

            compute(1)
            issue_out(o, 1)

        return carry

    lax.fori_loop(0, 20, pair_body, 0)
    wait_out(0, 0)
    wait_out(0, 1)


def kernel(values_l0, values_l1, values_l2, values_l3):
    return _ps_kernel(
        values_l0.reshape(N, KS[0]),
        values_l1.reshape(N, KS[1]),
        values_l2.reshape(N, KS[2]),
        values_l3.reshape(N, KS[3]),
    )


# speedup vs baseline: 1.1005x; 1.1005x over previous
import functools
import math

import jax
import jax.numpy as jnp
from jax import lax
from jax.experimental import pallas as pl
from jax.experimental.pallas import tpu as pltpu
from jax.experimental.pallas import tpu_sc as plsc

N = 20000
NQ = 16
MS = (1, 3, 5, 7)
KS = tuple(m * NQ for m in MS)
LOFF = (0, 256, 512, 768)
OUT_D = 1024
BR = 16
NBLK = N // BR
NW = 32
CG = tuple(1.0 / math.sqrt(2 * l + 1) for l in range(4))

_mesh = plsc.VectorSubcoreMesh(core_axis_name="c", subcore_axis_name="s")

_IN_SCRATCH = [
    pltpu.VMEM((BR, KS[l]), jnp.float32) for l in range(4)
]


@functools.partial(
    pl.kernel,
    mesh=_mesh,
    compiler_params=pltpu.CompilerParams(needs_layout_passes=False),
    out_type=jax.ShapeDtypeStruct((N, OUT_D), jnp.float32),
    scratch_types=[
        *_IN_SCRATCH,
        *_IN_SCRATCH,
        pltpu.VMEM((NQ, NQ), jnp.int32),
        pltpu.VMEM((BR, OUT_D), jnp.float32),
        pltpu.VMEM((BR, OUT_D), jnp.float32),
        pltpu.SemaphoreType.DMA,
        pltpu.SemaphoreType.DMA,
        pltpu.SemaphoreType.DMA,
        pltpu.SemaphoreType.DMA,
    ],
)
def _ps_kernel(v0, v1, v2, v3, out,
               a0, a1, a2, a3, c0, c1, c2, c3, idxtab, ob0, ob1,
               si0, si1, so0, so1):
    cid = lax.axis_index("c")
    sid = lax.axis_index("s")
    wid = sid * 2 + cid
    nblk_w = 39 + (wid < (NBLK - NW * (NBLK // NW))).astype(jnp.int32)

    vs = (v0, v1, v2, v3)
    ins = ((a0, a1, a2, a3), (c0, c1, c2, c3))
    obs = (ob0, ob1)
    sin = (si0, si1)
    sout = (so0, so1)

    def blk_r0(b):
        return (wid + b * NW) * BR

    def issue_in(b, s):
        for l in range(4):
            pltpu.async_copy(vs[l].at[pl.ds(blk_r0(b), BR)], ins[s][l], sin[s])

    def wait_in(b, s):
        for l in range(4):
            pltpu.make_async_copy(
                vs[l].at[pl.ds(blk_r0(b), BR)], ins[s][l], sin[s]
            ).wait()

    for q in range(NQ):
        idxtab[q] = jnp.full((NQ,), q, jnp.int32)

    def compute(s):
        bufs = ins[s]
        ob = obs[s]

        @plsc.parallel_loop(0, BR, unroll=2)
        def rbody(r):
            idxq = [idxtab[q] for q in range(NQ)]
            for l in range(4):
                M = MS[l]
                raw = [bufs[l][r, pl.ds(mm * NQ, NQ)] for mm in range(M)]
                if CG[l] != 1.0:
                    scv = [v * CG[l] for v in raw]
                else:
                    scv = raw
                npair = (M // 2) if l >= 2 else 0
                praw = [
                    plsc.pack(raw[2 * k], raw[2 * k + 1],
                              format=plsc.PackFormat.INTERLEAVED)
                    for k in range(npair)
                ]
                pw = [
                    plsc.bitcast(
                        plsc.pack(scv[2 * k], scv[2 * k + 1],
                                  format=plsc.PackFormat.INTERLEAVED),
                        jnp.int32,
                    )
                    for k in range(npair)
                ]
                for q in range(NQ):
                    terms = []
                    if npair:
                        pterms = [
                            plsc.bitcast(
                                jnp.take_along_axis(pw[k], idxq[q], axis=0),
                                jnp.bfloat16,
                            ) * praw[k]
                            for k in range(npair)
                        ]
                        pacc = pterms[0]
                        for k in range(1, npair):
                            pacc = pacc + pterms[k]
                        au, bu = plsc.unpack(
                            pacc, format=plsc.PackFormat.INTERLEAVED
                        )
                        terms = [au, bu]
                    for mm in range(2 * npair, M):
                        terms.append(
                            jnp.take_along_axis(scv[mm], idxq[q], axis=0)
                            * raw[mm]
                        )
                    while len(terms) > 1:
                        terms = [
                            terms[k] + terms[k + 1] if k + 1 < len(terms)
                            else terms[k]
                            for k in range(0, len(terms), 2)
                        ]
                    ob[r, pl.ds(LOFF[l] + q * NQ, NQ)] = terms[0]

    def issue_out(b, s):
        pltpu.async_copy(obs[s], out.at[pl.ds(blk_r0(b), BR)], sout[s])

    def wait_out(b, s):
        pltpu.make_async_copy(
            obs[s], out.at[pl.ds(blk_r0(b), BR)], sout[s]
        ).wait()

    issue_in(0, 0)

    def pair_body(i2, carry):
        e = 2 * i2
        o = e + 1

        @pl.when(o < nblk_w)
        def _():
            issue_in(o, 1)

        wait_in(e, 0)

        @pl.when(i2 > 0)
        def _():
            wait_out(e - 2, 0)

        compute(0)
        issue_out(e, 0)

        @pl.when(o < nblk_w)
        def _():
            @pl.when(o + 1 < nblk_w)
            def _():
                issue_in(o + 1, 0)

            wait_in(o, 1)

            @pl.when(i2 > 0)
            def _():
                wait_out(o - 2, 1)

            compute(1)
            issue_out(o, 1)

        return carry

    lax.fori_loop(0, 20, pair_body, 0)
    wait_out(0, 0)
    wait_out(0, 1)


def kernel(values_l0, values_l1, values_l2, values_l3):
    return _ps_kernel(
        values_l0.reshape(N, KS[0]),
        values_l1.reshape(N, KS[1]),
        values_l2.reshape(N, KS[2]),
        values_l3.reshape(N, KS[3]),
    )


# speedup vs baseline: 1.1132x; 1.0116x over previous
import functools
import math

import jax
import jax.numpy as jnp
from jax import lax
from jax.experimental import pallas as pl
from jax.experimental.pallas import tpu as pltpu
from jax.experimental.pallas import tpu_sc as plsc

N = 20000
NQ = 16
MS = (1, 3, 5, 7)
KS = tuple(m * NQ for m in MS)
LOFF = (0, 256, 512, 768)
OUT_D = 1024
BR = 16
NBLK = N // BR
NW = 32
CG = tuple(1.0 / math.sqrt(2 * l + 1) for l in range(4))

_mesh = plsc.VectorSubcoreMesh(core_axis_name="c", subcore_axis_name="s")

_IN_SCRATCH = [
    pltpu.VMEM((BR, KS[l]), jnp.float32) for l in range(4)
]


@functools.partial(
    pl.kernel,
    mesh=_mesh,
    compiler_params=pltpu.CompilerParams(needs_layout_passes=False),
    out_type=jax.ShapeDtypeStruct((N, OUT_D), jnp.float32),
    scratch_types=[
        *_IN_SCRATCH,
        *_IN_SCRATCH,
        pltpu.VMEM((NQ, NQ), jnp.int32),
        pltpu.VMEM((BR, OUT_D), jnp.float32),
        pltpu.VMEM((BR, OUT_D), jnp.float32),
        pltpu.SemaphoreType.DMA,
        pltpu.SemaphoreType.DMA,
        pltpu.SemaphoreType.DMA,
        pltpu.SemaphoreType.DMA,
    ],
)
def _ps_kernel(v0, v1, v2, v3, out,
               a0, a1, a2, a3, c0, c1, c2, c3, idxtab, ob0, ob1,
               si0, si1, so0, so1):
    cid = lax.axis_index("c")
    sid = lax.axis_index("s")
    wid = sid * 2 + cid
    nblk_w = 39 + (wid < (NBLK - NW * (NBLK // NW))).astype(jnp.int32)

    vs = (v0, v1, v2, v3)
    ins = ((a0, a1, a2, a3), (c0, c1, c2, c3))
    obs = (ob0, ob1)
    sin = (si0, si1)
    sout = (so0, so1)

    def blk_r0(b):
        return (wid + b * NW) * BR

    def issue_in(b, s):
        for l in range(4):
            pltpu.async_copy(vs[l].at[pl.ds(blk_r0(b), BR)], ins[s][l], sin[s])

    def wait_in(b, s):
        for l in range(4):
            pltpu.make_async_copy(
                vs[l].at[pl.ds(blk_r0(b), BR)], ins[s][l], sin[s]
            ).wait()

    for q in range(NQ):
        idxtab[q] = jnp.full((NQ,), q, jnp.int32)

    def compute(s):
        bufs = ins[s]
        ob = obs[s]

        @plsc.parallel_loop(0, BR, unroll=2)
        def rbody(r):
            idxq = [idxtab[q] for q in range(NQ)]
            for l in range(4):
                M = MS[l]
                raw = [bufs[l][r, pl.ds(mm * NQ, NQ)] for mm in range(M)]
                if CG[l] != 1.0:
                    scv = [v * CG[l] for v in raw]
                else:
                    scv = raw
                npair = (M // 2) if l >= 1 else 0
                praw = [
                    plsc.pack(raw[2 * k], raw[2 * k + 1],
                              format=plsc.PackFormat.INTERLEAVED)
                    for k in range(npair)
                ]
                pw = [
                    plsc.bitcast(
                        plsc.pack(scv[2 * k], scv[2 * k + 1],
                                  format=plsc.PackFormat.INTERLEAVED),
                        jnp.int32,
                    )
                    for k in range(npair)
                ]
                for q in range(NQ):
                    terms = []
                    if npair:
                        pterms = [
                            plsc.bitcast(
                                jnp.take_along_axis(pw[k], idxq[q], axis=0),
                                jnp.bfloat16,
                            ) * praw[k]
                            for k in range(npair)
                        ]
                        pacc = pterms[0]
                        for k in range(1, npair):
                            pacc = pacc + pterms[k]
                        au, bu = plsc.unpack(
                            pacc, format=plsc.PackFormat.INTERLEAVED
                        )
                        terms = [au, bu]
                    for mm in range(2 * npair, M):
                        terms.append(
                            jnp.take_along_axis(scv[mm], idxq[q], axis=0)
                            * raw[mm]
                        )
                    while len(terms) > 1:
                        terms = [
                            terms[k] + terms[k + 1] if k + 1 < len(terms)
                            else terms[k]
                            for k in range(0, len(terms), 2)
                        ]
                    ob[r, pl.ds(LOFF[l] + q * NQ, NQ)] = terms[0]

    def issue_out(b, s):
        pltpu.async_copy(obs[s], out.at[pl.ds(blk_r0(b), BR)], sout[s])

    def wait_out(b, s):
        pltpu.make_async_copy(
            obs[s], out.at[pl.ds(blk_r0(b), BR)], sout[s]
        ).wait()

    issue_in(0, 0)

    def pair_body(i2, carry):
        e = 2 * i2
        o = e + 1

        @pl.when(o < nblk_w)
        def _():
            issue_in(o, 1)

        wait_in(e, 0)

        @pl.when(i2 > 0)
        def _():
            wait_out(e - 2, 0)

        compute(0)
        issue_out(e, 0)

        @pl.when(o < nblk_w)
        def _():
            @pl.when(o + 1 < nblk_w)
            def _():
                issue_in(o + 1, 0)

            wait_in(o, 1)

            @pl.when(i2 > 0)
            def _():
                wait_out(o - 2, 1)

            compute(1)
            issue_out(o, 1)

        return carry

    lax.fori_loop(0, 20, pair_body, 0)
    wait_out(0, 0)
    wait_out(0, 1)


def kernel(values_l0, values_l1, values_l2, values_l3):
    return _ps_kernel(
        values_l0.reshape(N, KS[0]),
        values_l1.reshape(N, KS[1]),
        values_l2.reshape(N, KS[2]),
        values_l3.reshape(N, KS[3]),
    )
